# Initial kernel scaffold; baseline (speedup 1.0000x reference)
import functools

import jax
import jax.numpy as jnp
from jax import lax
from jax.experimental import pallas as pl
from jax.experimental.pallas import tpu as pltpu
from jax.experimental.pallas import tpu_sc as plsc

N_NODES = 10000
D = 128

NC = 2
NS = 16
NW = NC * NS
CHUNK = 128
NCH = 80
EPW = NCH * CHUNK
E_PAD = NW * EPW
N_PAD = 10240
ROWS_PER_TILE = N_PAD // NS
DUMMY_DST = N_NODES + 64
NB = 4

BM = 1280
GRID = N_PAD // BM


def _deg_kernel(dst_hbm, zeros_hbm, deg_out, dst_v, deg_v):
    cid = lax.axis_index("c")
    sid = lax.axis_index("s")
    wid = sid * NC + cid
    pltpu.sync_copy(dst_hbm.at[wid], dst_v)
    pltpu.sync_copy(zeros_hbm, deg_v)
    ones = jnp.ones((16,), jnp.float32)

    @pl.loop(0, NCH)
    def _(c):
        for g in range(CHUNK // 16):
            idx = dst_v[c, pl.ds(g * 16, 16)]
            plsc.addupdate_scatter(deg_v, [idx], ones)

    pltpu.sync_copy(deg_v, deg_out.at[wid])


@jax.jit
def _deg_call(dst_idx, zeros1d):
    mesh = plsc.VectorSubcoreMesh(core_axis_name="c", subcore_axis_name="s")
    return pl.kernel(
        _deg_kernel,
        out_type=jax.ShapeDtypeStruct((NW, N_PAD), jnp.float32),
        mesh=mesh,
        scratch_types=[
            pltpu.VMEM((NCH, CHUNK), jnp.int32),
            pltpu.VMEM((N_PAD,), jnp.float32),
        ],
    )(dst_idx, zeros1d)


def _edge_kernel(y_hbm, src_hbm, dst_hbm, zeros_hbm, p_out,
                 src_v, dst_v, buf, acc, gsem, ssem):
    cid = lax.axis_index("c")
    sid = lax.axis_index("s")
    pltpu.sync_copy(src_hbm.at[sid * NC + cid], src_v)
    pltpu.sync_copy(dst_hbm.at[sid * NC + cid], dst_v)
    r0 = sid * ROWS_PER_TILE
    pltpu.sync_copy(zeros_hbm.at[pl.ds(r0, ROWS_PER_TILE)],
                    acc.at[pl.ds(r0, ROWS_PER_TILE)])
    plsc.subcore_barrier()

    def g_start(c, b):
        pltpu.async_copy(y_hbm.at[src_v.at[c]], buf.at[b], gsem.at[b])

    def g_wait(c, b):
        pltpu.make_async_copy(y_hbm.at[src_v.at[c]], buf.at[b],
                              gsem.at[b]).wait()

    def s_start(c, b):
        pltpu.async_copy(buf.at[b], acc.at[dst_v.at[c]], ssem.at[b],
                         add=True)

    def s_wait(c, b):
        pltpu.make_async_copy(buf.at[b], acc.at[dst_v.at[c]],
                              ssem.at[b]).wait()

    for b in range(NB):
        g_start(b, b)

    @pl.loop(0, NCH - NB, step=NB)
    def _(j):
        for b in range(NB):
            c = j + b
            g_wait(c, b)
            s_start(c, b)
            s_wait(c, b)
            g_start(c + NB, b)

    for b in range(NB):
        c = NCH - NB + b
        g_wait(c, b)
        s_start(c, b)
        s_wait(c, b)

    plsc.subcore_barrier()
    pltpu.sync_copy(acc.at[pl.ds(r0, ROWS_PER_TILE)],
                    p_out.at[cid, pl.ds(r0, ROWS_PER_TILE)])


@jax.jit
def _edge_call(y, src_idx, dst_idx, zeros2d):
    mesh = plsc.VectorSubcoreMesh(core_axis_name="c", subcore_axis_name="s")
    return pl.kernel(
        _edge_kernel,
        out_type=jax.ShapeDtypeStruct((NC, N_PAD, D), jnp.float32),
        mesh=mesh,
        scratch_types=[
            pltpu.VMEM((NCH, CHUNK), jnp.int32),
            pltpu.VMEM((NCH, CHUNK), jnp.int32),
            pltpu.VMEM((NB, CHUNK, D), jnp.float32),
            pltpu.VMEM_SHARED((N_PAD, D), jnp.float32),
            pltpu.SemaphoreType.DMA((NB,)),
            pltpu.SemaphoreType.DMA((NB,)),
        ],
    )(y, src_idx, dst_idx, zeros2d)


def _mm_kernel(x_ref, w_ref, o_ref):
    o_ref[...] = jnp.dot(x_ref[...], w_ref[...],
                         preferred_element_type=jnp.float32)


def _mm_call(x, w):
    return pl.pallas_call(
        _mm_kernel,
        grid=(GRID,),
        in_specs=[
            pl.BlockSpec((BM, D), lambda i: (i, 0)),
            pl.BlockSpec((D, D), lambda i: (0, 0)),
        ],
        out_specs=pl.BlockSpec((BM, D), lambda i: (i, 0)),
        out_shape=jax.ShapeDtypeStruct((N_PAD, D), jnp.float32),
    )(x, w)


def _scale_kernel(degp_ref, xw_ref, dis_ref, y_ref):
    deg = jnp.sum(degp_ref[...], axis=0) + 1.0
    dis = lax.rsqrt(deg)[:, None]
    dis_ref[...] = dis
    y_ref[...] = xw_ref[...] * dis


def _scale_call(deg_parts, xw):
    return pl.pallas_call(
        _scale_kernel,
        grid=(GRID,),
        in_specs=[
            pl.BlockSpec((NW, BM), lambda i: (0, i)),
            pl.BlockSpec((BM, D), lambda i: (i, 0)),
        ],
        out_specs=[
            pl.BlockSpec((BM, 1), lambda i: (i, 0)),
            pl.BlockSpec((BM, D), lambda i: (i, 0)),
        ],
        out_shape=[
            jax.ShapeDtypeStruct((N_PAD, 1), jnp.float32),
            jax.ShapeDtypeStruct((N_PAD, D), jnp.float32),
        ],
    )(deg_parts, xw)


def _mid_kernel(p0_ref, p1_ref, xw_ref, dis_ref, b_ref, w_ref,
                xw2_ref, y2_ref):
    dis = dis_ref[...]
    xw = xw_ref[...]
    t = dis * (p0_ref[...] + p1_ref[...]) + (dis * dis) * xw + b_ref[...]
    h = jnp.maximum(t, 0.0)
    xw2 = jnp.dot(h, w_ref[...], preferred_element_type=jnp.float32)
    xw2_ref[...] = xw2
    y2_ref[...] = xw2 * dis


def _mid_call(p0, p1, xw, dis, b, w):
    return pl.pallas_call(
        _mid_kernel,
        grid=(GRID,),
        in_specs=[
            pl.BlockSpec((BM, D), lambda i: (i, 0)),
            pl.BlockSpec((BM, D), lambda i: (i, 0)),
            pl.BlockSpec((BM, D), lambda i: (i, 0)),
            pl.BlockSpec((BM, 1), lambda i: (i, 0)),
            pl.BlockSpec((1, D), lambda i: (0, 0)),
            pl.BlockSpec((D, D), lambda i: (0, 0)),
        ],
        out_specs=[
            pl.BlockSpec((BM, D), lambda i: (i, 0)),
            pl.BlockSpec((BM, D), lambda i: (i, 0)),
        ],
        out_shape=[
            jax.ShapeDtypeStruct((N_PAD, D), jnp.float32),
            jax.ShapeDtypeStruct((N_PAD, D), jnp.float32),
        ],
    )(p0, p1, xw, dis, b, w)


def _final_kernel(q0_ref, q1_ref, xw_ref, dis_ref, b_ref, w_ref, bl_ref,
                  o_ref):
    dis = dis_ref[...]
    t = (dis * (q0_ref[...] + q1_ref[...])
         + (dis * dis) * xw_ref[...] + b_ref[...])
    h = jnp.maximum(t, 0.0)
    o_ref[...] = jnp.dot(h, w_ref[...],
                         preferred_element_type=jnp.float32) + bl_ref[...]


def _final_call(q0, q1, xw2, dis, b2, wlin, blin):
    return pl.pallas_call(
        _final_kernel,
        grid=(GRID,),
        in_specs=[
            pl.BlockSpec((BM, D), lambda i: (i, 0)),
            pl.BlockSpec((BM, D), lambda i: (i, 0)),
            pl.BlockSpec((BM, D), lambda i: (i, 0)),
            pl.BlockSpec((BM, 1), lambda i: (i, 0)),
            pl.BlockSpec((1, D), lambda i: (0, 0)),
            pl.BlockSpec((D, D), lambda i: (0, 0)),
            pl.BlockSpec((1, D), lambda i: (0, 0)),
        ],
        out_specs=pl.BlockSpec((BM, D), lambda i: (i, 0)),
        out_shape=jax.ShapeDtypeStruct((N_PAD, D), jnp.float32),
    )(q0, q1, xw2, dis, b2, wlin, blin)


def kernel(x, edge_index, W1, b1, W2, b2, Wlin, blin):
    src = edge_index[0].astype(jnp.int32)
    dst = edge_index[1].astype(jnp.int32)
    n_extra = E_PAD - src.shape[0]
    src_p = jnp.concatenate([src, jnp.zeros((n_extra,), jnp.int32)])
    dst_p = jnp.concatenate(
        [dst, jnp.full((n_extra,), DUMMY_DST, jnp.int32)])
    src_idx = src_p.reshape(NW, NCH, CHUNK)
    dst_idx = dst_p.reshape(NW, NCH, CHUNK)

    x_pad = jnp.zeros((N_PAD, D), jnp.float32).at[:N_NODES].set(x)
    zeros1d = jnp.zeros((N_PAD,), jnp.float32)
    zeros2d = jnp.zeros((N_PAD, D), jnp.float32)

    deg_parts = _deg_call(dst_idx, zeros1d)
    xw1 = _mm_call(x_pad, W1)
    dis, y1 = _scale_call(deg_parts, xw1)

    p = _edge_call(y1, src_idx, dst_idx, zeros2d)
    xw2, y2 = _mid_call(p[0], p[1], xw1, dis, b1.reshape(1, D), W2)

    q = _edge_call(y2, src_idx, dst_idx, zeros2d)
    out = _final_call(q[0], q[1], xw2, dis, b2.reshape(1, D), Wlin,
                      blin.reshape(1, D))
    return out[:N_NODES]


# speedup vs baseline: 9.1602x; 9.1602x over previous
import functools

import jax
import jax.numpy as jnp
from jax import lax
from jax.experimental import pallas as pl
from jax.experimental.pallas import tpu as pltpu
from jax.experimental.pallas import tpu_sc as plsc

N_NODES = 10000
D = 128
HD = 64

NC = 2
NS = 16
NW = NC * NS
CHUNK = 128
NCH = 80
EPW = NCH * CHUNK
E_PAD = NW * EPW
N_PAD = 10240
ROWS_PER_TILE = N_PAD // NS
DUMMY_DST = N_NODES + 64
NB = 4

BM = 1280
GRID = N_PAD // BM


def _deg_kernel(dst_hbm, zeros_hbm, deg_out, dst_v, deg_v):
    cid = lax.axis_index("c")
    sid = lax.axis_index("s")
    wid = sid * NC + cid
    pltpu.sync_copy(dst_hbm.at[wid], dst_v)
    pltpu.sync_copy(zeros_hbm, deg_v)
    ones = jnp.ones((16,), jnp.float32)

    @pl.loop(0, NCH)
    def _(c):
        for g in range(CHUNK // 16):
            idx = dst_v[c, pl.ds(g * 16, 16)]
            plsc.addupdate_scatter(deg_v, [idx], ones)

    pltpu.sync_copy(deg_v, deg_out.at[wid])


@jax.jit
def _deg_call(dst_idx, zeros1d):
    mesh = plsc.VectorSubcoreMesh(core_axis_name="c", subcore_axis_name="s")
    return pl.kernel(
        _deg_kernel,
        out_type=jax.ShapeDtypeStruct((NW, N_PAD), jnp.float32),
        mesh=mesh,
        scratch_types=[
            pltpu.VMEM((NCH, CHUNK), jnp.int32),
            pltpu.VMEM((N_PAD,), jnp.float32),
        ],
        compiler_params=pltpu.CompilerParams(needs_layout_passes=False),
    )(dst_idx, zeros1d)


def _edge_kernel(ylo_hbm, yhi_hbm, src_hbm, dst_hbm, zeros_hbm,
                 plo_out, phi_out, src_v, dst_v, buf, acc, gsem, ssem):
    cid = lax.axis_index("c")
    sid = lax.axis_index("s")
    pltpu.sync_copy(src_hbm.at[sid * NC + cid], src_v)
    pltpu.sync_copy(dst_hbm.at[sid * NC + cid], dst_v)
    r0 = sid * ROWS_PER_TILE

    def g_start(y_hbm, c, b):
        pltpu.async_copy(y_hbm.at[src_v.at[c]], buf.at[b], gsem.at[b])

    def g_wait(y_hbm, c, b):
        pltpu.make_async_copy(y_hbm.at[src_v.at[c]], buf.at[b],
                              gsem.at[b]).wait()

    def s_start(c, b):
        pltpu.async_copy(buf.at[b], acc.at[dst_v.at[c]], ssem.at[b],
                         add=True)

    def s_wait(c, b):
        pltpu.make_async_copy(buf.at[b], acc.at[dst_v.at[c]],
                              ssem.at[b]).wait()

    for y_hbm, out_hbm in ((ylo_hbm, plo_out), (yhi_hbm, phi_out)):
        pltpu.sync_copy(zeros_hbm.at[pl.ds(r0, ROWS_PER_TILE)],
                        acc.at[pl.ds(r0, ROWS_PER_TILE)])
        plsc.subcore_barrier()

        for b in range(NB):
            g_start(y_hbm, b, b)

        @pl.loop(0, NCH - NB, step=NB)
        def _(j):
            for b in range(NB):
                c = j + b
                g_wait(y_hbm, c, b)
                s_start(c, b)
                s_wait(c, b)
                g_start(y_hbm, c + NB, b)

        for b in range(NB):
            c = NCH - NB + b
            g_wait(y_hbm, c, b)
            s_start(c, b)
            s_wait(c, b)

        plsc.subcore_barrier()
        pltpu.sync_copy(acc.at[pl.ds(r0, ROWS_PER_TILE)],
                        out_hbm.at[cid, pl.ds(r0, ROWS_PER_TILE)])


@jax.jit
def _edge_call(ylo, yhi, src_idx, dst_idx, zeros2d):
    mesh = plsc.VectorSubcoreMesh(core_axis_name="c", subcore_axis_name="s")
    return pl.kernel(
        _edge_kernel,
        out_type=(jax.ShapeDtypeStruct((NC, N_PAD, HD), jnp.float32),
                  jax.ShapeDtypeStruct((NC, N_PAD, HD), jnp.float32)),
        mesh=mesh,
        scratch_types=[
            pltpu.VMEM((NCH, CHUNK), jnp.int32),
            pltpu.VMEM((NCH, CHUNK), jnp.int32),
            pltpu.VMEM((NB, CHUNK, HD), jnp.float32),
            pltpu.VMEM_SHARED((N_PAD, HD), jnp.float32),
            pltpu.SemaphoreType.DMA((NB,)),
            pltpu.SemaphoreType.DMA((NB,)),
        ],
        compiler_params=pltpu.CompilerParams(needs_layout_passes=False,
                                             use_tc_tiling_on_sc=False),
    )(ylo, yhi, src_idx, dst_idx, zeros2d)


def _mm_kernel(x_ref, w_ref, o_ref):
    o_ref[...] = jnp.dot(x_ref[...], w_ref[...],
                         preferred_element_type=jnp.float32)


def _mm_call(x, w):
    return pl.pallas_call(
        _mm_kernel,
        grid=(GRID,),
        in_specs=[
            pl.BlockSpec((BM, D), lambda i: (i, 0)),
            pl.BlockSpec((D, D), lambda i: (0, 0)),
        ],
        out_specs=pl.BlockSpec((BM, D), lambda i: (i, 0)),
        out_shape=jax.ShapeDtypeStruct((N_PAD, D), jnp.float32),
    )(x, w)


def _scale_kernel(degp_ref, xw_ref, dis_ref, ylo_ref, yhi_ref):
    deg = jnp.sum(degp_ref[...], axis=0) + 1.0
    dis = lax.rsqrt(deg)[:, None]
    dis_ref[...] = dis
    y = xw_ref[...] * dis
    ylo_ref[...] = y[:, :HD]
    yhi_ref[...] = y[:, HD:]


def _scale_call(deg_parts, xw):
    return pl.pallas_call(
        _scale_kernel,
        grid=(GRID,),
        in_specs=[
            pl.BlockSpec((NW, BM), lambda i: (0, i)),
            pl.BlockSpec((BM, D), lambda i: (i, 0)),
        ],
        out_specs=[
            pl.BlockSpec((BM, 1), lambda i: (i, 0)),
            pl.BlockSpec((BM, HD), lambda i: (i, 0)),
            pl.BlockSpec((BM, HD), lambda i: (i, 0)),
        ],
        out_shape=[
            jax.ShapeDtypeStruct((N_PAD, 1), jnp.float32),
            jax.ShapeDtypeStruct((N_PAD, HD), jnp.float32),
            jax.ShapeDtypeStruct((N_PAD, HD), jnp.float32),
        ],
    )(deg_parts, xw)


def _mid_kernel(plo_ref, phi_ref, xw_ref, dis_ref, b_ref, w_ref,
                xw2_ref, y2lo_ref, y2hi_ref):
    dis = dis_ref[...]
    xw = xw_ref[...]
    plo = plo_ref[0] + plo_ref[1]
    phi = phi_ref[0] + phi_ref[1]
    p = jnp.concatenate([plo, phi], axis=1)
    t = dis * p + (dis * dis) * xw + b_ref[...]
    h = jnp.maximum(t, 0.0)
    xw2 = jnp.dot(h, w_ref[...], preferred_element_type=jnp.float32)
    xw2_ref[...] = xw2
    y2 = xw2 * dis
    y2lo_ref[...] = y2[:, :HD]
    y2hi_ref[...] = y2[:, HD:]


def _mid_call(plo, phi, xw, dis, b, w):
    return pl.pallas_call(
        _mid_kernel,
        grid=(GRID,),
        in_specs=[
            pl.BlockSpec((NC, BM, HD), lambda i: (0, i, 0)),
            pl.BlockSpec((NC, BM, HD), lambda i: (0, i, 0)),
            pl.BlockSpec((BM, D), lambda i: (i, 0)),
            pl.BlockSpec((BM, 1), lambda i: (i, 0)),
            pl.BlockSpec((1, D), lambda i: (0, 0)),
            pl.BlockSpec((D, D), lambda i: (0, 0)),
        ],
        out_specs=[
            pl.BlockSpec((BM, D), lambda i: (i, 0)),
            pl.BlockSpec((BM, HD), lambda i: (i, 0)),
            pl.BlockSpec((BM, HD), lambda i: (i, 0)),
        ],
        out_shape=[
            jax.ShapeDtypeStruct((N_PAD, D), jnp.float32),
            jax.ShapeDtypeStruct((N_PAD, HD), jnp.float32),
            jax.ShapeDtypeStruct((N_PAD, HD), jnp.float32),
        ],
    )(plo, phi, xw, dis, b, w)


def _final_kernel(qlo_ref, qhi_ref, xw_ref, dis_ref, b_ref, w_ref, bl_ref,
                  o_ref):
    dis = dis_ref[...]
    qlo = qlo_ref[0] + qlo_ref[1]
    qhi = qhi_ref[0] + qhi_ref[1]
    q = jnp.concatenate([qlo, qhi], axis=1)
    t = dis * q + (dis * dis) * xw_ref[...] + b_ref[...]
    h = jnp.maximum(t, 0.0)
    o_ref[...] = jnp.dot(h, w_ref[...],
                         preferred_element_type=jnp.float32) + bl_ref[...]


def _final_call(qlo, qhi, xw2, dis, b2, wlin, blin):
    return pl.pallas_call(
        _final_kernel,
        grid=(GRID,),
        in_specs=[
            pl.BlockSpec((NC, BM, HD), lambda i: (0, i, 0)),
            pl.BlockSpec((NC, BM, HD), lambda i: (0, i, 0)),
            pl.BlockSpec((BM, D), lambda i: (i, 0)),
            pl.BlockSpec((BM, 1), lambda i: (i, 0)),
            pl.BlockSpec((1, D), lambda i: (0, 0)),
            pl.BlockSpec((D, D), lambda i: (0, 0)),
            pl.BlockSpec((1, D), lambda i: (0, 0)),
        ],
        out_specs=pl.BlockSpec((BM, D), lambda i: (i, 0)),
        out_shape=jax.ShapeDtypeStruct((N_PAD, D), jnp.float32),
    )(qlo, qhi, xw2, dis, b2, wlin, blin)


def kernel(x, edge_index, W1, b1, W2, b2, Wlin, blin):
    src = edge_index[0].astype(jnp.int32)
    dst = edge_index[1].astype(jnp.int32)
    n_extra = E_PAD - src.shape[0]
    src_p = jnp.concatenate([src, jnp.zeros((n_extra,), jnp.int32)])
    dst_p = jnp.concatenate(
        [dst, jnp.full((n_extra,), DUMMY_DST, jnp.int32)])
    src_idx = src_p.reshape(NW, NCH, CHUNK)
    dst_idx = dst_p.reshape(NW, NCH, CHUNK)

    x_pad = jnp.zeros((N_PAD, D), jnp.float32).at[:N_NODES].set(x)
    zeros1d = jnp.zeros((N_PAD,), jnp.float32)
    zeros2d = jnp.zeros((N_PAD, HD), jnp.float32)

    deg_parts = _deg_call(dst_idx, zeros1d)
    xw1 = _mm_call(x_pad, W1)
    dis, y1lo, y1hi = _scale_call(deg_parts, xw1)

    plo, phi = _edge_call(y1lo, y1hi, src_idx, dst_idx, zeros2d)
    xw2, y2lo, y2hi = _mid_call(plo, phi, xw1, dis, b1.reshape(1, D), W2)

    qlo, qhi = _edge_call(y2lo, y2hi, src_idx, dst_idx, zeros2d)
    out = _final_call(qlo, qhi, xw2, dis, b2.reshape(1, D), Wlin,
                      blin.reshape(1, D))
    return out[:N_NODES]


# speedup vs baseline: 28.7233x; 3.1357x over previous
import functools

import jax
import jax.numpy as jnp
from jax import lax
from jax.experimental import pallas as pl
from jax.experimental.pallas import tpu as pltpu
from jax.experimental.pallas import tpu_sc as plsc

N_NODES = 10000
D = 128
HD = 64

NC = 2
NS = 16
NW = NC * NS
CHUNK = 128
NCH = 80
EPW = NCH * CHUNK
E_PAD = NW * EPW
N_PAD = 10240
ROWS_PER_TILE = N_PAD // NS
DUMMY_DST = N_NODES + 64
NB = 4

BM = 1280
GRID = N_PAD // BM


def _deg_kernel(dst_hbm, zeros_hbm, deg_out, dst_v, deg_v):
    cid = lax.axis_index("c")
    sid = lax.axis_index("s")
    wid = sid * NC + cid
    pltpu.sync_copy(dst_hbm.at[wid], dst_v)
    pltpu.sync_copy(zeros_hbm, deg_v)
    ones = jnp.ones((16,), jnp.float32)

    @pl.loop(0, NCH)
    def _(c):
        for g in range(CHUNK // 16):
            idx = dst_v[c, pl.ds(g * 16, 16)]
            plsc.addupdate_scatter(deg_v, [idx], ones)

    pltpu.sync_copy(deg_v, deg_out.at[wid])


@jax.jit
def _deg_call(dst_idx, zeros1d):
    mesh = plsc.VectorSubcoreMesh(core_axis_name="c", subcore_axis_name="s")
    return pl.kernel(
        _deg_kernel,
        out_type=jax.ShapeDtypeStruct((NW, N_PAD), jnp.float32),
        mesh=mesh,
        scratch_types=[
            pltpu.VMEM((NCH, CHUNK), jnp.int32),
            pltpu.VMEM((N_PAD,), jnp.float32),
        ],
        compiler_params=pltpu.CompilerParams(needs_layout_passes=False),
    )(dst_idx, zeros1d)


def _edge_kernel(ylo_hbm, yhi_hbm, src_hbm, dst_hbm, zeros_hbm,
                 plo_out, phi_out, src_v, dst_v, buf, acc, gsem, ssem):
    cid = lax.axis_index("c")
    sid = lax.axis_index("s")
    pltpu.sync_copy(src_hbm.at[sid * NC + cid], src_v)
    pltpu.sync_copy(dst_hbm.at[sid * NC + cid], dst_v)
    r0 = sid * ROWS_PER_TILE

    def g_start(y_hbm, c, b):
        pltpu.async_copy(y_hbm.at[src_v.at[c]], buf.at[b], gsem.at[b])

    def g_wait(y_hbm, c, b):
        pltpu.make_async_copy(y_hbm.at[src_v.at[c]], buf.at[b],
                              gsem.at[b]).wait()

    def s_start(c, b):
        pltpu.async_copy(buf.at[b], acc.at[dst_v.at[c]], ssem.at[b],
                         add=True)

    def s_wait(c, b):
        pltpu.make_async_copy(buf.at[b], acc.at[dst_v.at[c]],
                              ssem.at[b]).wait()

    for y_hbm, out_hbm in ((ylo_hbm, plo_out), (yhi_hbm, phi_out)):
        pltpu.sync_copy(zeros_hbm.at[pl.ds(r0, ROWS_PER_TILE)],
                        acc.at[pl.ds(r0, ROWS_PER_TILE)])
        plsc.subcore_barrier()

        for b in range(NB):
            g_start(y_hbm, b, b)

        @pl.loop(0, NCH - NB, step=NB)
        def _(j):
            for b in range(NB):
                c = j + b
                g_wait(y_hbm, c, b)
                s_start(c, b)
                s_wait(c, b)
                g_start(y_hbm, c + NB, b)

        for b in range(NB):
            c = NCH - NB + b
            g_wait(y_hbm, c, b)
            s_start(c, b)
            s_wait(c, b)

        plsc.subcore_barrier()
        pltpu.sync_copy(acc.at[pl.ds(r0, ROWS_PER_TILE)],
                        out_hbm.at[cid, pl.ds(r0, ROWS_PER_TILE)])


@jax.jit
def _edge_call(ylo, yhi, src_idx, dst_idx, zeros2d):
    mesh = plsc.VectorSubcoreMesh(core_axis_name="c", subcore_axis_name="s")
    return pl.kernel(
        _edge_kernel,
        out_type=(jax.ShapeDtypeStruct((NC, N_PAD, HD), jnp.float32),
                  jax.ShapeDtypeStruct((NC, N_PAD, HD), jnp.float32)),
        mesh=mesh,
        scratch_types=[
            pltpu.VMEM((NCH, CHUNK), jnp.int32),
            pltpu.VMEM((NCH, CHUNK), jnp.int32),
            pltpu.VMEM((NB, CHUNK, HD), jnp.float32),
            pltpu.VMEM_SHARED((N_PAD, HD), jnp.float32),
            pltpu.SemaphoreType.DMA((NB,)),
            pltpu.SemaphoreType.DMA((NB,)),
        ],
        compiler_params=pltpu.CompilerParams(needs_layout_passes=False,
                                             use_tc_tiling_on_sc=False),
    )(ylo, yhi, src_idx, dst_idx, zeros2d)


def _mm_kernel(x_ref, w_ref, o_ref):
    o_ref[...] = jnp.dot(x_ref[...], w_ref[...],
                         preferred_element_type=jnp.float32)


def _mm_call(x, w):
    return pl.pallas_call(
        _mm_kernel,
        grid=(GRID,),
        in_specs=[
            pl.BlockSpec((BM, D), lambda i: (i, 0)),
            pl.BlockSpec((D, D), lambda i: (0, 0)),
        ],
        out_specs=pl.BlockSpec((BM, D), lambda i: (i, 0)),
        out_shape=jax.ShapeDtypeStruct((N_PAD, D), jnp.float32),
    )(x, w)


def _scale_kernel(degp_ref, xw_ref, dis_ref, ylo_ref, yhi_ref):
    deg = jnp.sum(degp_ref[...], axis=0) + 1.0
    dis = lax.rsqrt(deg)[:, None]
    dis_ref[...] = dis
    y = xw_ref[...] * dis
    ylo_ref[...] = y[:, :HD]
    yhi_ref[...] = y[:, HD:]


def _scale_call(deg_parts, xw):
    return pl.pallas_call(
        _scale_kernel,
        grid=(GRID,),
        in_specs=[
            pl.BlockSpec((NW, BM), lambda i: (0, i)),
            pl.BlockSpec((BM, D), lambda i: (i, 0)),
        ],
        out_specs=[
            pl.BlockSpec((BM, 1), lambda i: (i, 0)),
            pl.BlockSpec((BM, HD), lambda i: (i, 0)),
            pl.BlockSpec((BM, HD), lambda i: (i, 0)),
        ],
        out_shape=[
            jax.ShapeDtypeStruct((N_PAD, 1), jnp.float32),
            jax.ShapeDtypeStruct((N_PAD, HD), jnp.float32),
            jax.ShapeDtypeStruct((N_PAD, HD), jnp.float32),
        ],
    )(deg_parts, xw)


def _mid_kernel(plo_ref, phi_ref, xw_ref, dis_ref, b_ref, w_ref,
                xw2_ref, y2lo_ref, y2hi_ref):
    dis = dis_ref[...]
    xw = xw_ref[...]
    plo = plo_ref[0] + plo_ref[1]
    phi = phi_ref[0] + phi_ref[1]
    p = jnp.concatenate([plo, phi], axis=1)
    t = dis * p + (dis * dis) * xw + b_ref[...]
    h = jnp.maximum(t, 0.0)
    xw2 = jnp.dot(h, w_ref[...], preferred_element_type=jnp.float32)
    xw2_ref[...] = xw2
    y2 = xw2 * dis
    y2lo_ref[...] = y2[:, :HD]
    y2hi_ref[...] = y2[:, HD:]


def _mid_call(plo, phi, xw, dis, b, w):
    return pl.pallas_call(
        _mid_kernel,
        grid=(GRID,),
        in_specs=[
            pl.BlockSpec((NC, BM, HD), lambda i: (0, i, 0)),
            pl.BlockSpec((NC, BM, HD), lambda i: (0, i, 0)),
            pl.BlockSpec((BM, D), lambda i: (i, 0)),
            pl.BlockSpec((BM, 1), lambda i: (i, 0)),
            pl.BlockSpec((1, D), lambda i: (0, 0)),
            pl.BlockSpec((D, D), lambda i: (0, 0)),
        ],
        out_specs=[
            pl.BlockSpec((BM, D), lambda i: (i, 0)),
            pl.BlockSpec((BM, HD), lambda i: (i, 0)),
            pl.BlockSpec((BM, HD), lambda i: (i, 0)),
        ],
        out_shape=[
            jax.ShapeDtypeStruct((N_PAD, D), jnp.float32),
            jax.ShapeDtypeStruct((N_PAD, HD), jnp.float32),
            jax.ShapeDtypeStruct((N_PAD, HD), jnp.float32),
        ],
    )(plo, phi, xw, dis, b, w)


def _final_kernel(qlo_ref, qhi_ref, xw_ref, dis_ref, b_ref, w_ref, bl_ref,
                  o_ref):
    dis = dis_ref[...]
    qlo = qlo_ref[0] + qlo_ref[1]
    qhi = qhi_ref[0] + qhi_ref[1]
    q = jnp.concatenate([qlo, qhi], axis=1)
    t = dis * q + (dis * dis) * xw_ref[...] + b_ref[...]
    h = jnp.maximum(t, 0.0)
    o_ref[...] = jnp.dot(h, w_ref[...],
                         preferred_element_type=jnp.float32) + bl_ref[...]


def _final_call(qlo, qhi, xw2, dis, b2, wlin, blin):
    return pl.pallas_call(
        _final_kernel,
        grid=(GRID,),
        in_specs=[
            pl.BlockSpec((NC, BM, HD), lambda i: (0, i, 0)),
            pl.BlockSpec((NC, BM, HD), lambda i: (0, i, 0)),
            pl.BlockSpec((BM, D), lambda i: (i, 0)),
            pl.BlockSpec((BM, 1), lambda i: (i, 0)),
            pl.BlockSpec((1, D), lambda i: (0, 0)),
            pl.BlockSpec((D, D), lambda i: (0, 0)),
            pl.BlockSpec((1, D), lambda i: (0, 0)),
        ],
        out_specs=pl.BlockSpec((BM, D), lambda i: (i, 0)),
        out_shape=jax.ShapeDtypeStruct((N_PAD, D), jnp.float32),
    )(qlo, qhi, xw2, dis, b2, wlin, blin)


def kernel(x, edge_index, W1, b1, W2, b2, Wlin, blin):
    src = edge_index[0].astype(jnp.int32)
    dst = edge_index[1].astype(jnp.int32)
    epw_real = src.shape[0] // NW
    n_extra = EPW - epw_real
    pad_src = jnp.broadcast_to(jnp.arange(n_extra, dtype=jnp.int32),
                               (NW, n_extra))
    pad_dst = jnp.broadcast_to(
        N_NODES + (jnp.arange(n_extra, dtype=jnp.int32) % (N_PAD - N_NODES)),
        (NW, n_extra))
    src_idx = jnp.concatenate(
        [src.reshape(NW, epw_real), pad_src], axis=1).reshape(NW, NCH, CHUNK)
    dst_idx = jnp.concatenate(
        [dst.reshape(NW, epw_real), pad_dst], axis=1).reshape(NW, NCH, CHUNK)

    x_pad = jnp.zeros((N_PAD, D), jnp.float32).at[:N_NODES].set(x)
    zeros1d = jnp.zeros((N_PAD,), jnp.float32)
    zeros2d = jnp.zeros((N_PAD, HD), jnp.float32)

    deg_parts = _deg_call(dst_idx, zeros1d)
    xw1 = _mm_call(x_pad, W1)
    dis, y1lo, y1hi = _scale_call(deg_parts, xw1)

    plo, phi = _edge_call(y1lo, y1hi, src_idx, dst_idx, zeros2d)
    xw2, y2lo, y2hi = _mid_call(plo, phi, xw1, dis, b1.reshape(1, D), W2)

    qlo, qhi = _edge_call(y2lo, y2hi, src_idx, dst_idx, zeros2d)
    out = _final_call(qlo, qhi, xw2, dis, b2.reshape(1, D), Wlin,
                      blin.reshape(1, D))
    return out[:N_NODES]


# speedup vs baseline: 32.0230x; 1.1149x over previous
import functools

import jax
import jax.numpy as jnp
from jax import lax
from jax.experimental import pallas as pl
from jax.experimental.pallas import tpu as pltpu
from jax.experimental.pallas import tpu_sc as plsc

N_NODES = 10000
D = 128
HD = 64

NC = 2
NS = 16
NW = NC * NS
CHUNK = 128
NCH = 160
EPW = NCH * CHUNK
E_PAD = NS * EPW
N_PAD = 10240
ROWS_PER_TILE = N_PAD // NS
DUMMY_DST = N_NODES + 64
NB = 4

BM = 1280
GRID = N_PAD // BM


def _deg_kernel(dst_hbm, zeros_hbm, deg_out, dst_v, deg_v):
    cid = lax.axis_index("c")
    sid = lax.axis_index("s")
    wid = sid * NC + cid
    pltpu.sync_copy(dst_hbm.at[sid, pl.ds(cid * (NCH // NC), NCH // NC)],
                    dst_v)
    pltpu.sync_copy(zeros_hbm, deg_v)
    ones = jnp.ones((16,), jnp.float32)

    @pl.loop(0, NCH // NC)
    def _(c):
        for g in range(CHUNK // 16):
            idx = dst_v[c, pl.ds(g * 16, 16)]
            plsc.addupdate_scatter(deg_v, [idx], ones)

    pltpu.sync_copy(deg_v, deg_out.at[wid])


@jax.jit
def _deg_call(dst_idx, zeros1d):
    mesh = plsc.VectorSubcoreMesh(core_axis_name="c", subcore_axis_name="s")
    return pl.kernel(
        _deg_kernel,
        out_type=jax.ShapeDtypeStruct((NW, N_PAD), jnp.float32),
        mesh=mesh,
        scratch_types=[
            pltpu.VMEM((NCH // NC, CHUNK), jnp.int32),
            pltpu.VMEM((N_PAD,), jnp.float32),
        ],
        compiler_params=pltpu.CompilerParams(needs_layout_passes=False),
    )(dst_idx, zeros1d)


def _edge_kernel(ylo_hbm, yhi_hbm, src_hbm, dst_hbm, zeros_hbm,
                 plo_out, phi_out, src_v, dst_v, buf, acc, gsem, ssem):
    cid = lax.axis_index("c")
    sid = lax.axis_index("s")
    pltpu.sync_copy(src_hbm.at[sid], src_v)
    pltpu.sync_copy(dst_hbm.at[sid], dst_v)
    r0 = sid * ROWS_PER_TILE
    pltpu.sync_copy(zeros_hbm.at[pl.ds(r0, ROWS_PER_TILE)],
                    acc.at[pl.ds(r0, ROWS_PER_TILE)])
    plsc.subcore_barrier()

    def phase(y_hbm, out_hbm):
        def g_start(c, b):
            pltpu.async_copy(y_hbm.at[src_v.at[c]], buf.at[b], gsem.at[b])

        def g_wait(c, b):
            pltpu.make_async_copy(y_hbm.at[src_v.at[c]], buf.at[b],
                                  gsem.at[b]).wait()

        def s_start(c, b):
            pltpu.async_copy(buf.at[b], acc.at[dst_v.at[c]], ssem.at[b],
                             add=True)

        def s_wait(c, b):
            pltpu.make_async_copy(buf.at[b], acc.at[dst_v.at[c]],
                                  ssem.at[b]).wait()

        for b in range(NB):
            g_start(b, b)

        @pl.loop(0, NCH - NB, step=NB)
        def _(j):
            for b in range(NB):
                c = j + b
                g_wait(c, b)
                s_start(c, b)
                s_wait(c, b)
                g_start(c + NB, b)

        for b in range(NB):
            c = NCH - NB + b
            g_wait(c, b)
            s_start(c, b)
            s_wait(c, b)

        plsc.subcore_barrier()
        pltpu.sync_copy(acc.at[pl.ds(r0, ROWS_PER_TILE)],
                        out_hbm.at[pl.ds(r0, ROWS_PER_TILE)])

    @pl.when(cid == 0)
    def _():
        phase(ylo_hbm, plo_out)

    @pl.when(cid == 1)
    def _():
        phase(yhi_hbm, phi_out)


@jax.jit
def _edge_call(ylo, yhi, src_idx, dst_idx, zeros2d):
    mesh = plsc.VectorSubcoreMesh(core_axis_name="c", subcore_axis_name="s")
    return pl.kernel(
        _edge_kernel,
        out_type=(jax.ShapeDtypeStruct((N_PAD, HD), jnp.float32),
                  jax.ShapeDtypeStruct((N_PAD, HD), jnp.float32)),
        mesh=mesh,
        scratch_types=[
            pltpu.VMEM((NCH, CHUNK), jnp.int32),
            pltpu.VMEM((NCH, CHUNK), jnp.int32),
            pltpu.VMEM((NB, CHUNK, HD), jnp.float32),
            pltpu.VMEM_SHARED((N_PAD, HD), jnp.float32),
            pltpu.SemaphoreType.DMA((NB,)),
            pltpu.SemaphoreType.DMA((NB,)),
        ],
        compiler_params=pltpu.CompilerParams(needs_layout_passes=False,
                                             use_tc_tiling_on_sc=False),
    )(ylo, yhi, src_idx, dst_idx, zeros2d)


def _mm_kernel(x_ref, w_ref, o_ref):
    o_ref[...] = jnp.dot(x_ref[...], w_ref[...],
                         preferred_element_type=jnp.float32)


def _mm_call(x, w):
    return pl.pallas_call(
        _mm_kernel,
        grid=(GRID,),
        in_specs=[
            pl.BlockSpec((BM, D), lambda i: (i, 0)),
            pl.BlockSpec((D, D), lambda i: (0, 0)),
        ],
        out_specs=pl.BlockSpec((BM, D), lambda i: (i, 0)),
        out_shape=jax.ShapeDtypeStruct((N_PAD, D), jnp.float32),
    )(x, w)


def _scale_kernel(degp_ref, xw_ref, dis_ref, ylo_ref, yhi_ref):
    deg = jnp.sum(degp_ref[...], axis=0) + 1.0
    dis = lax.rsqrt(deg)[:, None]
    dis_ref[...] = dis
    y = xw_ref[...] * dis
    ylo_ref[...] = y[:, :HD]
    yhi_ref[...] = y[:, HD:]


def _scale_call(deg_parts, xw):
    return pl.pallas_call(
        _scale_kernel,
        grid=(GRID,),
        in_specs=[
            pl.BlockSpec((NW, BM), lambda i: (0, i)),
            pl.BlockSpec((BM, D), lambda i: (i, 0)),
        ],
        out_specs=[
            pl.BlockSpec((BM, 1), lambda i: (i, 0)),
            pl.BlockSpec((BM, HD), lambda i: (i, 0)),
            pl.BlockSpec((BM, HD), lambda i: (i, 0)),
        ],
        out_shape=[
            jax.ShapeDtypeStruct((N_PAD, 1), jnp.float32),
            jax.ShapeDtypeStruct((N_PAD, HD), jnp.float32),
            jax.ShapeDtypeStruct((N_PAD, HD), jnp.float32),
        ],
    )(deg_parts, xw)


def _mid_kernel(plo_ref, phi_ref, xw_ref, dis_ref, b_ref, w_ref,
                xw2_ref, y2lo_ref, y2hi_ref):
    dis = dis_ref[...]
    xw = xw_ref[...]
    p = jnp.concatenate([plo_ref[...], phi_ref[...]], axis=1)
    t = dis * p + (dis * dis) * xw + b_ref[...]
    h = jnp.maximum(t, 0.0)
    xw2 = jnp.dot(h, w_ref[...], preferred_element_type=jnp.float32)
    xw2_ref[...] = xw2
    y2 = xw2 * dis
    y2lo_ref[...] = y2[:, :HD]
    y2hi_ref[...] = y2[:, HD:]


def _mid_call(plo, phi, xw, dis, b, w):
    return pl.pallas_call(
        _mid_kernel,
        grid=(GRID,),
        in_specs=[
            pl.BlockSpec((BM, HD), lambda i: (i, 0)),
            pl.BlockSpec((BM, HD), lambda i: (i, 0)),
            pl.BlockSpec((BM, D), lambda i: (i, 0)),
            pl.BlockSpec((BM, 1), lambda i: (i, 0)),
            pl.BlockSpec((1, D), lambda i: (0, 0)),
            pl.BlockSpec((D, D), lambda i: (0, 0)),
        ],
        out_specs=[
            pl.BlockSpec((BM, D), lambda i: (i, 0)),
            pl.BlockSpec((BM, HD), lambda i: (i, 0)),
            pl.BlockSpec((BM, HD), lambda i: (i, 0)),
        ],
        out_shape=[
            jax.ShapeDtypeStruct((N_PAD, D), jnp.float32),
            jax.ShapeDtypeStruct((N_PAD, HD), jnp.float32),
            jax.ShapeDtypeStruct((N_PAD, HD), jnp.float32),
        ],
    )(plo, phi, xw, dis, b, w)


def _final_kernel(qlo_ref, qhi_ref, xw_ref, dis_ref, b_ref, w_ref, bl_ref,
                  o_ref):
    dis = dis_ref[...]
    q = jnp.concatenate([qlo_ref[...], qhi_ref[...]], axis=1)
    t = dis * q + (dis * dis) * xw_ref[...] + b_ref[...]
    h = jnp.maximum(t, 0.0)
    o_ref[...] = jnp.dot(h, w_ref[...],
                         preferred_element_type=jnp.float32) + bl_ref[...]


def _final_call(qlo, qhi, xw2, dis, b2, wlin, blin):
    return pl.pallas_call(
        _final_kernel,
        grid=(GRID,),
        in_specs=[
            pl.BlockSpec((BM, HD), lambda i: (i, 0)),
            pl.BlockSpec((BM, HD), lambda i: (i, 0)),
            pl.BlockSpec((BM, D), lambda i: (i, 0)),
            pl.BlockSpec((BM, 1), lambda i: (i, 0)),
            pl.BlockSpec((1, D), lambda i: (0, 0)),
            pl.BlockSpec((D, D), lambda i: (0, 0)),
            pl.BlockSpec((1, D), lambda i: (0, 0)),
        ],
        out_specs=pl.BlockSpec((BM, D), lambda i: (i, 0)),
        out_shape=jax.ShapeDtypeStruct((N_PAD, D), jnp.float32),
    )(qlo, qhi, xw2, dis, b2, wlin, blin)


def kernel(x, edge_index, W1, b1, W2, b2, Wlin, blin):
    src = edge_index[0].astype(jnp.int32)
    dst = edge_index[1].astype(jnp.int32)
    epw_real = src.shape[0] // NS
    n_extra = EPW - epw_real
    pad_src = jnp.broadcast_to(jnp.arange(n_extra, dtype=jnp.int32),
                               (NS, n_extra))
    pad_dst = jnp.broadcast_to(
        N_NODES + (jnp.arange(n_extra, dtype=jnp.int32) % (N_PAD - N_NODES)),
        (NS, n_extra))
    src_idx = jnp.concatenate(
        [src.reshape(NS, epw_real), pad_src], axis=1).reshape(NS, NCH, CHUNK)
    dst_idx = jnp.concatenate(
        [dst.reshape(NS, epw_real), pad_dst], axis=1).reshape(NS, NCH, CHUNK)

    x_pad = jnp.zeros((N_PAD, D), jnp.float32).at[:N_NODES].set(x)
    zeros1d = jnp.zeros((N_PAD,), jnp.float32)
    zeros2d = jnp.zeros((N_PAD, HD), jnp.float32)

    deg_parts = _deg_call(dst_idx, zeros1d)
    xw1 = _mm_call(x_pad, W1)
    dis, y1lo, y1hi = _scale_call(deg_parts, xw1)

    plo, phi = _edge_call(y1lo, y1hi, src_idx, dst_idx, zeros2d)
    xw2, y2lo, y2hi = _mid_call(plo, phi, xw1, dis, b1.reshape(1, D), W2)

    qlo, qhi = _edge_call(y2lo, y2hi, src_idx, dst_idx, zeros2d)
    out = _final_call(qlo, qhi, xw2, dis, b2.reshape(1, D), Wlin,
                      blin.reshape(1, D))
    return out[:N_NODES]


# speedup vs baseline: 33.7745x; 1.0547x over previous
import functools

import jax
import jax.numpy as jnp
from jax import lax
from jax.experimental import pallas as pl
from jax.experimental.pallas import tpu as pltpu
from jax.experimental.pallas import tpu_sc as plsc

N_NODES = 10000
D = 128
HD = 64

NC = 2
NS = 16
NW = NC * NS
CHUNK = 128
NCH = 160
EPW = NCH * CHUNK
E_PAD = NS * EPW
N_PAD = 10240
ROWS_PER_TILE = N_PAD // NS
DUMMY_DST = N_NODES + 64
NB = 4

BM = 1280
GRID = N_PAD // BM
BMF = 1000


def _deg_kernel(idx_hbm, zeros_hbm, deg_out, dst_v, deg_v):
    cid = lax.axis_index("c")
    sid = lax.axis_index("s")
    wid = sid * NC + cid
    pltpu.sync_copy(idx_hbm.at[1, sid, pl.ds(cid * (NCH // NC), NCH // NC)],
                    dst_v)
    pltpu.sync_copy(zeros_hbm, deg_v)
    ones = jnp.ones((16,), jnp.float32)

    @pl.loop(0, NCH // NC)
    def _(c):
        for g in range(CHUNK // 16):
            idx = dst_v[c, pl.ds(g * 16, 16)]
            plsc.addupdate_scatter(deg_v, [idx], ones)

    pltpu.sync_copy(deg_v, deg_out.at[wid])


@jax.jit
def _deg_call(idx, zeros1d):
    mesh = plsc.VectorSubcoreMesh(core_axis_name="c", subcore_axis_name="s")
    return pl.kernel(
        _deg_kernel,
        out_type=jax.ShapeDtypeStruct((NW, N_PAD), jnp.float32),
        mesh=mesh,
        scratch_types=[
            pltpu.VMEM((NCH // NC, CHUNK), jnp.int32),
            pltpu.VMEM((N_PAD,), jnp.float32),
        ],
        compiler_params=pltpu.CompilerParams(needs_layout_passes=False,
                                             use_tc_tiling_on_sc=False),
    )(idx, zeros1d)


def _edge_kernel(ylo_hbm, yhi_hbm, idx_hbm, zeros_hbm,
                 plo_out, phi_out, src_v, dst_v, buf, acc, gsem, ssem):
    cid = lax.axis_index("c")
    sid = lax.axis_index("s")
    pltpu.sync_copy(idx_hbm.at[0, sid], src_v)
    pltpu.sync_copy(idx_hbm.at[1, sid], dst_v)
    r0 = sid * ROWS_PER_TILE
    pltpu.sync_copy(zeros_hbm.at[pl.ds(r0, ROWS_PER_TILE)],
                    acc.at[pl.ds(r0, ROWS_PER_TILE)])
    plsc.subcore_barrier()

    def phase(y_hbm, out_hbm):
        def g_start(c, b):
            pltpu.async_copy(y_hbm.at[src_v.at[c]], buf.at[b], gsem.at[b])

        def g_wait(c, b):
            pltpu.make_async_copy(y_hbm.at[src_v.at[c]], buf.at[b],
                                  gsem.at[b]).wait()

        def s_start(c, b):
            pltpu.async_copy(buf.at[b], acc.at[dst_v.at[c]], ssem.at[b],
                             add=True)

        def s_wait(c, b):
            pltpu.make_async_copy(buf.at[b], acc.at[dst_v.at[c]],
                                  ssem.at[b]).wait()

        for b in range(NB):
            g_start(b, b)

        @pl.loop(0, NCH - NB, step=NB)
        def _(j):
            for b in range(NB):
                c = j + b
                g_wait(c, b)
                s_start(c, b)
                s_wait(c, b)
                g_start(c + NB, b)

        for b in range(NB):
            c = NCH - NB + b
            g_wait(c, b)
            s_start(c, b)
            s_wait(c, b)

        plsc.subcore_barrier()
        pltpu.sync_copy(acc.at[pl.ds(r0, ROWS_PER_TILE)],
                        out_hbm.at[pl.ds(r0, ROWS_PER_TILE)])

    @pl.when(cid == 0)
    def _():
        phase(ylo_hbm, plo_out)

    @pl.when(cid == 1)
    def _():
        phase(yhi_hbm, phi_out)


@jax.jit
def _edge_call(ylo, yhi, idx, zeros2d):
    mesh = plsc.VectorSubcoreMesh(core_axis_name="c", subcore_axis_name="s")
    return pl.kernel(
        _edge_kernel,
        out_type=(jax.ShapeDtypeStruct((N_PAD, HD), jnp.float32),
                  jax.ShapeDtypeStruct((N_PAD, HD), jnp.float32)),
        mesh=mesh,
        scratch_types=[
            pltpu.VMEM((NCH, CHUNK), jnp.int32),
            pltpu.VMEM((NCH, CHUNK), jnp.int32),
            pltpu.VMEM((NB, CHUNK, HD), jnp.float32),
            pltpu.VMEM_SHARED((N_PAD, HD), jnp.float32),
            pltpu.SemaphoreType.DMA((NB,)),
            pltpu.SemaphoreType.DMA((NB,)),
        ],
        compiler_params=pltpu.CompilerParams(needs_layout_passes=False,
                                             use_tc_tiling_on_sc=False),
    )(ylo, yhi, idx, zeros2d)


def _pre_kernel(x_ref, w_ref, degp_ref, xw_ref, dis_ref, ylo_ref, yhi_ref):
    xw = jnp.dot(x_ref[...], w_ref[...], preferred_element_type=jnp.float32)
    xw_ref[...] = xw
    deg = jnp.sum(degp_ref[...], axis=0) + 1.0
    dis = lax.rsqrt(deg)[:, None]
    dis_ref[...] = dis
    y = xw * dis
    ylo_ref[...] = y[:, :HD]
    yhi_ref[...] = y[:, HD:]


def _pre_call(x, w, deg_parts):
    return pl.pallas_call(
        _pre_kernel,
        grid=(GRID,),
        in_specs=[
            pl.BlockSpec((BM, D), lambda i: (i, 0)),
            pl.BlockSpec((D, D), lambda i: (0, 0)),
            pl.BlockSpec((NW, BM), lambda i: (0, i)),
        ],
        out_specs=[
            pl.BlockSpec((BM, D), lambda i: (i, 0)),
            pl.BlockSpec((BM, 1), lambda i: (i, 0)),
            pl.BlockSpec((BM, HD), lambda i: (i, 0)),
            pl.BlockSpec((BM, HD), lambda i: (i, 0)),
        ],
        out_shape=[
            jax.ShapeDtypeStruct((N_PAD, D), jnp.float32),
            jax.ShapeDtypeStruct((N_PAD, 1), jnp.float32),
            jax.ShapeDtypeStruct((N_PAD, HD), jnp.float32),
            jax.ShapeDtypeStruct((N_PAD, HD), jnp.float32),
        ],
    )(x, w, deg_parts)


def _mid_kernel(plo_ref, phi_ref, xw_ref, dis_ref, b_ref, w_ref,
                xw2_ref, y2lo_ref, y2hi_ref):
    dis = dis_ref[...]
    xw = xw_ref[...]
    p = jnp.concatenate([plo_ref[...], phi_ref[...]], axis=1)
    t = dis * p + (dis * dis) * xw + b_ref[...]
    h = jnp.maximum(t, 0.0)
    xw2 = jnp.dot(h, w_ref[...], preferred_element_type=jnp.float32)
    xw2_ref[...] = xw2
    y2 = xw2 * dis
    y2lo_ref[...] = y2[:, :HD]
    y2hi_ref[...] = y2[:, HD:]


def _mid_call(plo, phi, xw, dis, b, w):
    return pl.pallas_call(
        _mid_kernel,
        grid=(GRID,),
        in_specs=[
            pl.BlockSpec((BM, HD), lambda i: (i, 0)),
            pl.BlockSpec((BM, HD), lambda i: (i, 0)),
            pl.BlockSpec((BM, D), lambda i: (i, 0)),
            pl.BlockSpec((BM, 1), lambda i: (i, 0)),
            pl.BlockSpec((1, D), lambda i: (0, 0)),
            pl.BlockSpec((D, D), lambda i: (0, 0)),
        ],
        out_specs=[
            pl.BlockSpec((BM, D), lambda i: (i, 0)),
            pl.BlockSpec((BM, HD), lambda i: (i, 0)),
            pl.BlockSpec((BM, HD), lambda i: (i, 0)),
        ],
        out_shape=[
            jax.ShapeDtypeStruct((N_PAD, D), jnp.float32),
            jax.ShapeDtypeStruct((N_PAD, HD), jnp.float32),
            jax.ShapeDtypeStruct((N_PAD, HD), jnp.float32),
        ],
    )(plo, phi, xw, dis, b, w)


def _final_kernel(qlo_ref, qhi_ref, xw_ref, dis_ref, b_ref, w_ref, bl_ref,
                  o_ref):
    dis = dis_ref[...]
    q = jnp.concatenate([qlo_ref[...], qhi_ref[...]], axis=1)
    t = dis * q + (dis * dis) * xw_ref[...] + b_ref[...]
    h = jnp.maximum(t, 0.0)
    o_ref[...] = jnp.dot(h, w_ref[...],
                         preferred_element_type=jnp.float32) + bl_ref[...]


def _final_call(qlo, qhi, xw2, dis, b2, wlin, blin):
    return pl.pallas_call(
        _final_kernel,
        grid=(N_NODES // BMF,),
        in_specs=[
            pl.BlockSpec((BMF, HD), lambda i: (i, 0)),
            pl.BlockSpec((BMF, HD), lambda i: (i, 0)),
            pl.BlockSpec((BMF, D), lambda i: (i, 0)),
            pl.BlockSpec((BMF, 1), lambda i: (i, 0)),
            pl.BlockSpec((1, D), lambda i: (0, 0)),
            pl.BlockSpec((D, D), lambda i: (0, 0)),
            pl.BlockSpec((1, D), lambda i: (0, 0)),
        ],
        out_specs=pl.BlockSpec((BMF, D), lambda i: (i, 0)),
        out_shape=jax.ShapeDtypeStruct((N_NODES, D), jnp.float32),
    )(qlo, qhi, xw2, dis, b2, wlin, blin)


def kernel(x, edge_index, W1, b1, W2, b2, Wlin, blin):
    e = edge_index.shape[1]
    epw_real = e // NS
    n_extra = EPW - epw_real
    pad_src = jnp.broadcast_to(jnp.arange(n_extra, dtype=jnp.int32),
                               (1, NS, n_extra))
    pad_dst = jnp.broadcast_to(
        N_NODES + (jnp.arange(n_extra, dtype=jnp.int32) % (N_PAD - N_NODES)),
        (1, NS, n_extra))
    idx = jnp.concatenate(
        [edge_index.astype(jnp.int32).reshape(2, NS, epw_real),
         jnp.concatenate([pad_src, pad_dst], axis=0)],
        axis=2).reshape(2, NS, NCH, CHUNK)

    x_pad = jnp.zeros((N_PAD, D), jnp.float32).at[:N_NODES].set(x)
    zeros1d = jnp.zeros((N_PAD,), jnp.float32)
    zeros2d = jnp.zeros((N_PAD, HD), jnp.float32)

    deg_parts = _deg_call(idx, zeros1d)
    xw1, dis, y1lo, y1hi = _pre_call(x_pad, W1, deg_parts)

    plo, phi = _edge_call(y1lo, y1hi, idx, zeros2d)
    xw2, y2lo, y2hi = _mid_call(plo, phi, xw1, dis, b1.reshape(1, D), W2)

    qlo, qhi = _edge_call(y2lo, y2hi, idx, zeros2d)
    return _final_call(qlo, qhi, xw2, dis, b2.reshape(1, D), Wlin,
                       blin.reshape(1, D))


# speedup vs baseline: 34.4240x; 1.0192x over previous
import functools

import jax
import jax.numpy as jnp
from jax import lax
from jax.experimental import pallas as pl
from jax.experimental.pallas import tpu as pltpu
from jax.experimental.pallas import tpu_sc as plsc

N_NODES = 10000
D = 128
HD = 64

NC = 2
NS = 16
NW = NC * NS
CHUNK = 128
NCH = 160
EPW = NCH * CHUNK
E_PAD = NS * EPW
N_PAD = 10240
ROWS_PER_TILE = N_PAD // NS
DUMMY_DST = N_NODES + 64
NB = 4

BM = 1280
GRID = N_PAD // BM
BMF = 1000


def _deg_kernel(idx_hbm, deg_out, dst_v, deg_v):
    cid = lax.axis_index("c")
    sid = lax.axis_index("s")
    wid = sid * NC + cid
    pltpu.sync_copy(idx_hbm.at[1, sid, pl.ds(cid * (NCH // NC), NCH // NC)],
                    dst_v)
    zeros = jnp.zeros((16,), jnp.float32)

    @pl.loop(0, N_PAD // 16)
    def _(r):
        deg_v[pl.ds(r * 16, 16)] = zeros

    ones = jnp.ones((16,), jnp.float32)

    @pl.loop(0, NCH // NC)
    def _(c):
        for g in range(CHUNK // 16):
            idx = dst_v[c, pl.ds(g * 16, 16)]
            plsc.addupdate_scatter(deg_v, [idx], ones)

    pltpu.sync_copy(deg_v, deg_out.at[wid])


@jax.jit
def _deg_call(idx):
    mesh = plsc.VectorSubcoreMesh(core_axis_name="c", subcore_axis_name="s")
    return pl.kernel(
        _deg_kernel,
        out_type=jax.ShapeDtypeStruct((NW, N_PAD), jnp.float32),
        mesh=mesh,
        scratch_types=[
            pltpu.VMEM((NCH // NC, CHUNK), jnp.int32),
            pltpu.VMEM((N_PAD,), jnp.float32),
        ],
        compiler_params=pltpu.CompilerParams(needs_layout_passes=False,
                                             use_tc_tiling_on_sc=False),
    )(idx)


def _edge_kernel(ylo_hbm, yhi_hbm, idx_hbm,
                 plo_out, phi_out, src_v, dst_v, buf, acc, gsem, ssem):
    cid = lax.axis_index("c")
    sid = lax.axis_index("s")
    pltpu.sync_copy(idx_hbm.at[0, sid], src_v)
    pltpu.sync_copy(idx_hbm.at[1, sid], dst_v)
    r0 = sid * ROWS_PER_TILE
    zeros = jnp.zeros((16,), jnp.float32)

    @pl.loop(0, CHUNK)
    def _(r):
        for g in range(HD // 16):
            buf[0, r, pl.ds(g * 16, 16)] = zeros

    for k in range(ROWS_PER_TILE // CHUNK):
        pltpu.sync_copy(buf.at[0], acc.at[pl.ds(r0 + k * CHUNK, CHUNK)])
    plsc.subcore_barrier()

    def phase(y_hbm, out_hbm):
        def g_start(c, b):
            pltpu.async_copy(y_hbm.at[src_v.at[c]], buf.at[b], gsem.at[b])

        def g_wait(c, b):
            pltpu.make_async_copy(y_hbm.at[src_v.at[c]], buf.at[b],
                                  gsem.at[b]).wait()

        def s_start(c, b):
            pltpu.async_copy(buf.at[b], acc.at[dst_v.at[c]], ssem.at[b],
                             add=True)

        def s_wait(c, b):
            pltpu.make_async_copy(buf.at[b], acc.at[dst_v.at[c]],
                                  ssem.at[b]).wait()

        for b in range(NB):
            g_start(b, b)

        @pl.loop(0, NCH - NB, step=NB)
        def _(j):
            for b in range(NB):
                c = j + b
                g_wait(c, b)
                s_start(c, b)
                s_wait(c, b)
                g_start(c + NB, b)

        for b in range(NB):
            c = NCH - NB + b
            g_wait(c, b)
            s_start(c, b)
            s_wait(c, b)

        plsc.subcore_barrier()
        pltpu.sync_copy(acc.at[pl.ds(r0, ROWS_PER_TILE)],
                        out_hbm.at[pl.ds(r0, ROWS_PER_TILE)])

    @pl.when(cid == 0)
    def _():
        phase(ylo_hbm, plo_out)

    @pl.when(cid == 1)
    def _():
        phase(yhi_hbm, phi_out)


@jax.jit
def _edge_call(ylo, yhi, idx):
    mesh = plsc.VectorSubcoreMesh(core_axis_name="c", subcore_axis_name="s")
    return pl.kernel(
        _edge_kernel,
        out_type=(jax.ShapeDtypeStruct((N_PAD, HD), jnp.float32),
                  jax.ShapeDtypeStruct((N_PAD, HD), jnp.float32)),
        mesh=mesh,
        scratch_types=[
            pltpu.VMEM((NCH, CHUNK), jnp.int32),
            pltpu.VMEM((NCH, CHUNK), jnp.int32),
            pltpu.VMEM((NB, CHUNK, HD), jnp.float32),
            pltpu.VMEM_SHARED((N_PAD, HD), jnp.float32),
            pltpu.SemaphoreType.DMA((NB,)),
            pltpu.SemaphoreType.DMA((NB,)),
        ],
        compiler_params=pltpu.CompilerParams(needs_layout_passes=False,
                                             use_tc_tiling_on_sc=False),
    )(ylo, yhi, idx)


def _pre_kernel(x_ref, w_ref, degp_ref, xw_ref, dis_ref, ylo_ref, yhi_ref):
    xw = jnp.dot(x_ref[...], w_ref[...], preferred_element_type=jnp.float32)
    xw_ref[...] = xw
    deg = jnp.sum(degp_ref[...], axis=0) + 1.0
    dis = lax.rsqrt(deg)[:, None]
    dis_ref[...] = dis
    y = xw * dis
    ylo_ref[...] = y[:, :HD]
    yhi_ref[...] = y[:, HD:]


def _pre_call(x, w, deg_parts):
    return pl.pallas_call(
        _pre_kernel,
        grid=(GRID,),
        in_specs=[
            pl.BlockSpec((BM, D), lambda i: (i, 0)),
            pl.BlockSpec((D, D), lambda i: (0, 0)),
            pl.BlockSpec((NW, BM), lambda i: (0, i)),
        ],
        out_specs=[
            pl.BlockSpec((BM, D), lambda i: (i, 0)),
            pl.BlockSpec((BM, 1), lambda i: (i, 0)),
            pl.BlockSpec((BM, HD), lambda i: (i, 0)),
            pl.BlockSpec((BM, HD), lambda i: (i, 0)),
        ],
        out_shape=[
            jax.ShapeDtypeStruct((N_PAD, D), jnp.float32),
            jax.ShapeDtypeStruct((N_PAD, 1), jnp.float32),
            jax.ShapeDtypeStruct((N_PAD, HD), jnp.float32),
            jax.ShapeDtypeStruct((N_PAD, HD), jnp.float32),
        ],
    )(x, w, deg_parts)


def _mid_kernel(plo_ref, phi_ref, xw_ref, dis_ref, b_ref, w_ref,
                xw2_ref, y2lo_ref, y2hi_ref):
    dis = dis_ref[...]
    xw = xw_ref[...]
    p = jnp.concatenate([plo_ref[...], phi_ref[...]], axis=1)
    t = dis * p + (dis * dis) * xw + b_ref[...]
    h = jnp.maximum(t, 0.0)
    xw2 = jnp.dot(h, w_ref[...], preferred_element_type=jnp.float32)
    xw2_ref[...] = xw2
    y2 = xw2 * dis
    y2lo_ref[...] = y2[:, :HD]
    y2hi_ref[...] = y2[:, HD:]


def _mid_call(plo, phi, xw, dis, b, w):
    return pl.pallas_call(
        _mid_kernel,
        grid=(GRID,),
        in_specs=[
            pl.BlockSpec((BM, HD), lambda i: (i, 0)),
            pl.BlockSpec((BM, HD), lambda i: (i, 0)),
            pl.BlockSpec((BM, D), lambda i: (i, 0)),
            pl.BlockSpec((BM, 1), lambda i: (i, 0)),
            pl.BlockSpec((1, D), lambda i: (0, 0)),
            pl.BlockSpec((D, D), lambda i: (0, 0)),
        ],
        out_specs=[
            pl.BlockSpec((BM, D), lambda i: (i, 0)),
            pl.BlockSpec((BM, HD), lambda i: (i, 0)),
            pl.BlockSpec((BM, HD), lambda i: (i, 0)),
        ],
        out_shape=[
            jax.ShapeDtypeStruct((N_PAD, D), jnp.float32),
            jax.ShapeDtypeStruct((N_PAD, HD), jnp.float32),
            jax.ShapeDtypeStruct((N_PAD, HD), jnp.float32),
        ],
    )(plo, phi, xw, dis, b, w)


def _final_kernel(qlo_ref, qhi_ref, xw_ref, dis_ref, b_ref, w_ref, bl_ref,
                  o_ref):
    dis = dis_ref[...]
    q = jnp.concatenate([qlo_ref[...], qhi_ref[...]], axis=1)
    t = dis * q + (dis * dis) * xw_ref[...] + b_ref[...]
    h = jnp.maximum(t, 0.0)
    o_ref[...] = jnp.dot(h, w_ref[...],
                         preferred_element_type=jnp.float32) + bl_ref[...]


def _final_call(qlo, qhi, xw2, dis, b2, wlin, blin):
    return pl.pallas_call(
        _final_kernel,
        grid=(N_NODES // BMF,),
        in_specs=[
            pl.BlockSpec((BMF, HD), lambda i: (i, 0)),
            pl.BlockSpec((BMF, HD), lambda i: (i, 0)),
            pl.BlockSpec((BMF, D), lambda i: (i, 0)),
            pl.BlockSpec((BMF, 1), lambda i: (i, 0)),
            pl.BlockSpec((1, D), lambda i: (0, 0)),
            pl.BlockSpec((D, D), lambda i: (0, 0)),
            pl.BlockSpec((1, D), lambda i: (0, 0)),
        ],
        out_specs=pl.BlockSpec((BMF, D), lambda i: (i, 0)),
        out_shape=jax.ShapeDtypeStruct((N_NODES, D), jnp.float32),
    )(qlo, qhi, xw2, dis, b2, wlin, blin)


def kernel(x, edge_index, W1, b1, W2, b2, Wlin, blin):
    e = edge_index.shape[1]
    epw_real = e // NS
    n_extra = EPW - epw_real
    pad_src = jnp.broadcast_to(jnp.arange(n_extra, dtype=jnp.int32),
                               (1, NS, n_extra))
    pad_dst = jnp.broadcast_to(
        N_NODES + (jnp.arange(n_extra, dtype=jnp.int32) % (N_PAD - N_NODES)),
        (1, NS, n_extra))
    idx = jnp.concatenate(
        [edge_index.astype(jnp.int32).reshape(2, NS, epw_real),
         jnp.concatenate([pad_src, pad_dst], axis=0)],
        axis=2).reshape(2, NS, NCH, CHUNK)

    x_pad = jnp.zeros((N_PAD, D), jnp.float32).at[:N_NODES].set(x)

    deg_parts = _deg_call(idx)
    xw1, dis, y1lo, y1hi = _pre_call(x_pad, W1, deg_parts)

    plo, phi = _edge_call(y1lo, y1hi, idx)
    xw2, y2lo, y2hi = _mid_call(plo, phi, xw1, dis, b1.reshape(1, D), W2)

    qlo, qhi = _edge_call(y2lo, y2hi, idx)
    return _final_call(qlo, qhi, xw2, dis, b2.reshape(1, D), Wlin,
                       blin.reshape(1, D))


# speedup vs baseline: 36.7482x; 1.0675x over previous
import functools

import jax
import jax.numpy as jnp
from jax import lax
from jax.experimental import pallas as pl
from jax.experimental.pallas import tpu as pltpu
from jax.experimental.pallas import tpu_sc as plsc

N_NODES = 10000
D = 128
HD = 64

NC = 2
NS = 16
NW = NC * NS
CHUNK = 128
NCH = 160
EPW = NCH * CHUNK
E_PAD = NS * EPW
N_PAD = 10240
ROWS_PER_TILE = N_PAD // NS
DUMMY_DST = N_NODES + 64
NB = 4

BM = 1280
GRID = N_PAD // BM
BMF = 1000


def _deg_kernel(idx_hbm, deg_out, dst_v, deg_v):
    cid = lax.axis_index("c")
    sid = lax.axis_index("s")
    wid = sid * NC + cid
    pltpu.sync_copy(idx_hbm.at[1, sid, pl.ds(cid * (NCH // NC), NCH // NC)],
                    dst_v)
    zeros = jnp.zeros((16,), jnp.float32)

    @pl.loop(0, N_PAD // 16)
    def _(r):
        deg_v[pl.ds(r * 16, 16)] = zeros

    ones = jnp.ones((16,), jnp.float32)

    @pl.loop(0, NCH // NC)
    def _(c):
        for g in range(CHUNK // 16):
            idx = dst_v[c, pl.ds(g * 16, 16)]
            plsc.addupdate_scatter(deg_v, [idx], ones)

    pltpu.sync_copy(deg_v, deg_out.at[wid])


@jax.jit
def _deg_call(idx):
    mesh = plsc.VectorSubcoreMesh(core_axis_name="c", subcore_axis_name="s")
    return pl.kernel(
        _deg_kernel,
        out_type=jax.ShapeDtypeStruct((NW, N_PAD), jnp.float32),
        mesh=mesh,
        scratch_types=[
            pltpu.VMEM((NCH // NC, CHUNK), jnp.int32),
            pltpu.VMEM((N_PAD,), jnp.float32),
        ],
        compiler_params=pltpu.CompilerParams(needs_layout_passes=False,
                                             use_tc_tiling_on_sc=False),
    )(idx)


def _edge_kernel(ylo_hbm, yhi_hbm, idx_hbm, p_out,
                 src_v, dst_v, buf, acc, gsem, ssem):
    cid = lax.axis_index("c")
    sid = lax.axis_index("s")
    pltpu.sync_copy(idx_hbm.at[0, sid], src_v)
    pltpu.sync_copy(idx_hbm.at[1, sid], dst_v)
    r0 = sid * ROWS_PER_TILE
    zeros = jnp.zeros((16,), jnp.float32)

    @pl.loop(0, CHUNK)
    def _(r):
        for g in range(HD // 16):
            buf[0, r, pl.ds(g * 16, 16)] = zeros

    for k in range(ROWS_PER_TILE // CHUNK):
        pltpu.sync_copy(buf.at[0], acc.at[pl.ds(r0 + k * CHUNK, CHUNK)])
    plsc.subcore_barrier()

    def phase(y_hbm, col0):
        def g_start(c, b):
            pltpu.async_copy(y_hbm.at[src_v.at[c]], buf.at[b], gsem.at[b])

        def g_wait(c, b):
            pltpu.make_async_copy(y_hbm.at[src_v.at[c]], buf.at[b],
                                  gsem.at[b]).wait()

        def s_start(c, b):
            pltpu.async_copy(buf.at[b], acc.at[dst_v.at[c]], ssem.at[b],
                             add=True)

        def s_wait(c, b):
            pltpu.make_async_copy(buf.at[b], acc.at[dst_v.at[c]],
                                  ssem.at[b]).wait()

        for b in range(NB):
            g_start(b, b)

        @pl.loop(0, NCH - NB, step=NB)
        def _(j):
            for b in range(NB):
                c = j + b
                g_wait(c, b)
                s_start(c, b)
                s_wait(c, b)
                g_start(c + NB, b)

        for b in range(NB):
            c = NCH - NB + b
            g_wait(c, b)
            s_start(c, b)
            s_wait(c, b)

        plsc.subcore_barrier()
        pltpu.sync_copy(
            acc.at[pl.ds(r0, ROWS_PER_TILE)],
            p_out.at[pl.ds(r0, ROWS_PER_TILE), pl.ds(col0, HD)])

    @pl.when(cid == 0)
    def _():
        phase(ylo_hbm, 0)

    @pl.when(cid == 1)
    def _():
        phase(yhi_hbm, HD)


@jax.jit
def _edge_call(ylo, yhi, idx):
    mesh = plsc.VectorSubcoreMesh(core_axis_name="c", subcore_axis_name="s")
    return pl.kernel(
        _edge_kernel,
        out_type=jax.ShapeDtypeStruct((N_PAD, D), jnp.float32),
        mesh=mesh,
        scratch_types=[
            pltpu.VMEM((NCH, CHUNK), jnp.int32),
            pltpu.VMEM((NCH, CHUNK), jnp.int32),
            pltpu.VMEM((NB, CHUNK, HD), jnp.float32),
            pltpu.VMEM_SHARED((N_PAD, HD), jnp.float32),
            pltpu.SemaphoreType.DMA((NB,)),
            pltpu.SemaphoreType.DMA((NB,)),
        ],
        compiler_params=pltpu.CompilerParams(needs_layout_passes=False,
                                             use_tc_tiling_on_sc=False),
    )(ylo, yhi, idx)


def _pre_kernel(x_ref, w_ref, degp_ref, xw_ref, dis_ref, ylo_ref, yhi_ref):
    xw = jnp.dot(x_ref[...], w_ref[...], preferred_element_type=jnp.float32)
    xw_ref[...] = xw
    deg = jnp.sum(degp_ref[...], axis=0) + 1.0
    dis = lax.rsqrt(deg)[:, None]
    dis_ref[...] = dis
    y = xw * dis
    ylo_ref[...] = y[:, :HD]
    yhi_ref[...] = y[:, HD:]


def _pre_call(x, w, deg_parts):
    return pl.pallas_call(
        _pre_kernel,
        grid=(GRID,),
        in_specs=[
            pl.BlockSpec((BM, D), lambda i: (i, 0)),
            pl.BlockSpec((D, D), lambda i: (0, 0)),
            pl.BlockSpec((NW, BM), lambda i: (0, i)),
        ],
        out_specs=[
            pl.BlockSpec((BM, D), lambda i: (i, 0)),
            pl.BlockSpec((BM, 1), lambda i: (i, 0)),
            pl.BlockSpec((BM, HD), lambda i: (i, 0)),
            pl.BlockSpec((BM, HD), lambda i: (i, 0)),
        ],
        out_shape=[
            jax.ShapeDtypeStruct((N_PAD, D), jnp.float32),
            jax.ShapeDtypeStruct((N_PAD, 1), jnp.float32),
            jax.ShapeDtypeStruct((N_PAD, HD), jnp.float32),
            jax.ShapeDtypeStruct((N_PAD, HD), jnp.float32),
        ],
    )(x, w, deg_parts)


def _mid_kernel(p_ref, xw_ref, dis_ref, b_ref, w_ref,
                xw2_ref, y2lo_ref, y2hi_ref):
    dis = dis_ref[...]
    xw = xw_ref[...]
    t = dis * p_ref[...] + (dis * dis) * xw + b_ref[...]
    h = jnp.maximum(t, 0.0)
    xw2 = jnp.dot(h, w_ref[...], preferred_element_type=jnp.float32)
    xw2_ref[...] = xw2
    y2 = xw2 * dis
    y2lo_ref[...] = y2[:, :HD]
    y2hi_ref[...] = y2[:, HD:]


def _mid_call(p, xw, dis, b, w):
    return pl.pallas_call(
        _mid_kernel,
        grid=(GRID,),
        in_specs=[
            pl.BlockSpec((BM, D), lambda i: (i, 0)),
            pl.BlockSpec((BM, D), lambda i: (i, 0)),
            pl.BlockSpec((BM, 1), lambda i: (i, 0)),
            pl.BlockSpec((1, D), lambda i: (0, 0)),
            pl.BlockSpec((D, D), lambda i: (0, 0)),
        ],
        out_specs=[
            pl.BlockSpec((BM, D), lambda i: (i, 0)),
            pl.BlockSpec((BM, HD), lambda i: (i, 0)),
            pl.BlockSpec((BM, HD), lambda i: (i, 0)),
        ],
        out_shape=[
            jax.ShapeDtypeStruct((N_PAD, D), jnp.float32),
            jax.ShapeDtypeStruct((N_PAD, HD), jnp.float32),
            jax.ShapeDtypeStruct((N_PAD, HD), jnp.float32),
        ],
    )(p, xw, dis, b, w)


def _final_kernel(q_ref, xw_ref, dis_ref, b_ref, w_ref, bl_ref, o_ref):
    dis = dis_ref[...]
    t = dis * q_ref[...] + (dis * dis) * xw_ref[...] + b_ref[...]
    h = jnp.maximum(t, 0.0)
    o_ref[...] = jnp.dot(h, w_ref[...],
                         preferred_element_type=jnp.float32) + bl_ref[...]


def _final_call(q, xw2, dis, b2, wlin, blin):
    return pl.pallas_call(
        _final_kernel,
        grid=(N_NODES // BMF,),
        in_specs=[
            pl.BlockSpec((BMF, D), lambda i: (i, 0)),
            pl.BlockSpec((BMF, D), lambda i: (i, 0)),
            pl.BlockSpec((BMF, 1), lambda i: (i, 0)),
            pl.BlockSpec((1, D), lambda i: (0, 0)),
            pl.BlockSpec((D, D), lambda i: (0, 0)),
            pl.BlockSpec((1, D), lambda i: (0, 0)),
        ],
        out_specs=pl.BlockSpec((BMF, D), lambda i: (i, 0)),
        out_shape=jax.ShapeDtypeStruct((N_NODES, D), jnp.float32),
    )(q, xw2, dis, b2, wlin, blin)


def kernel(x, edge_index, W1, b1, W2, b2, Wlin, blin):
    e = edge_index.shape[1]
    epw_real = e // NS
    n_extra = EPW - epw_real
    pad_src = jnp.broadcast_to(jnp.arange(n_extra, dtype=jnp.int32),
                               (1, NS, n_extra))
    pad_dst = jnp.broadcast_to(
        N_NODES + (jnp.arange(n_extra, dtype=jnp.int32) % (N_PAD - N_NODES)),
        (1, NS, n_extra))
    idx = jnp.concatenate(
        [edge_index.astype(jnp.int32).reshape(2, NS, epw_real),
         jnp.concatenate([pad_src, pad_dst], axis=0)],
        axis=2).reshape(2, NS, NCH, CHUNK)

    x_pad = jnp.zeros((N_PAD, D), jnp.float32).at[:N_NODES].set(x)

    deg_parts = _deg_call(idx)
    xw1, dis, y1lo, y1hi = _pre_call(x_pad, W1, deg_parts)

    p = _edge_call(y1lo, y1hi, idx)
    xw2, y2lo, y2hi = _mid_call(p, xw1, dis, b1.reshape(1, D), W2)

    q = _edge_call(y2lo, y2hi, idx)
    return _final_call(q, xw2, dis, b2.reshape(1, D), Wlin,
                       blin.reshape(1, D))
